# baseline (speedup 1.0000x reference)
import functools

import jax
import jax.numpy as jnp
from jax import lax
from jax.experimental import pallas as pl
from jax.experimental.pallas import tpu as pltpu
from jax.experimental.pallas import tpu_sc as plsc

HIDDEN_HALF = 1024
MAX_RES = 4096
MAX_CHAINS = 64
N_TOKENS = 16384

_info = plsc.get_sparse_core_info()
NC = _info.num_cores
NS = _info.num_subcores
L = _info.num_lanes
NW = NC * NS
B_PER_T = N_TOKENS // NW
CHUNK = 32
N_CHUNKS = B_PER_T // CHUNK
NBUF = 3

_mesh = plsc.VectorSubcoreMesh(core_axis_name="c", subcore_axis_name="s")


@functools.partial(
    pl.kernel,
    mesh=_mesh,
    out_type=jax.ShapeDtypeStruct((N_TOKENS, 2 * HIDDEN_HALF), jnp.float32),
    scratch_types=[
        pltpu.VMEM((B_PER_T,), jnp.int32),
        pltpu.VMEM((B_PER_T,), jnp.int32),
        pltpu.VMEM((CHUNK, HIDDEN_HALF), jnp.float32),
        pltpu.VMEM((CHUNK, HIDDEN_HALF), jnp.float32),
        pltpu.VMEM((CHUNK, HIDDEN_HALF), jnp.float32),
        pltpu.VMEM_SHARED((MAX_CHAINS, HIDDEN_HALF), jnp.float32),
        pltpu.SemaphoreType.DMA,
        pltpu.SemaphoreType.DMA,
        pltpu.SemaphoreType.DMA,
    ],
)
def _embed_kernel(res_idx_hbm, chain_idx_hbm, res_tab_hbm, chain_tab_hbm,
                  out_hbm, ridx_v, cidx_v, buf0, buf1, buf2, chain_sp,
                  gsem, wsem, csem):
    core = lax.axis_index("c")
    sub = lax.axis_index("s")
    wid = sub * NC + core
    base = wid * B_PER_T
    bufs = (buf0, buf1, buf2)

    @pl.when(sub == 0)
    def _():
        pltpu.sync_copy(chain_tab_hbm, chain_sp)

    pltpu.sync_copy(res_idx_hbm.at[pl.ds(base, B_PER_T)], ridx_v)
    pltpu.sync_copy(chain_idx_hbm.at[pl.ds(base, B_PER_T)], cidx_v)

    def _clip(i, carry):
        sl = pl.ds(i * L, L)
        ridx_v[sl] = jnp.clip(ridx_v[sl] - 1, 0, MAX_RES - 1)
        cidx_v[sl] = jnp.clip(cidx_v[sl], 0, MAX_CHAINS - 1)
        return carry

    lax.fori_loop(0, B_PER_T // L, _clip, 0)
    plsc.subcore_barrier()

    def res_gather(c, buf):
        isl = pl.ds(c * CHUNK, CHUNK)
        return pltpu.async_copy(res_tab_hbm.at[ridx_v.at[isl]], buf, gsem)

    g = [None] * NBUF
    w = [None] * NBUF
    ch = []
    for p in range(min(2, N_CHUNKS)):
        g[p] = res_gather(p, bufs[p])
    for grp in range(B_PER_T // L):
        rows = cidx_v[pl.ds(grp * L, L)]
        for lane in range(L):
            t = grp * L + lane
            ch.append(pltpu.async_copy(
                chain_sp.at[rows[lane]],
                out_hbm.at[base + t, pl.ds(HIDDEN_HALF, HIDDEN_HALF)],
                csem))
    for c in range(N_CHUNKS):
        cur = c % NBUF
        g[cur].wait()
        if c + 2 < N_CHUNKS:
            nxt = (c + 2) % NBUF
            if w[nxt] is not None:
                w[nxt].wait()
            g[nxt] = res_gather(c + 2, bufs[nxt])
        w[cur] = pltpu.async_copy(
            bufs[cur],
            out_hbm.at[pl.ds(base + c * CHUNK, CHUNK), pl.ds(0, HIDDEN_HALF)],
            wsem)
    for p in range(NBUF):
        if w[p] is not None:
            w[p].wait()
    for h in ch:
        h.wait()


def kernel(residue_index, asym_id, residue_embed, chain_embed):
    return _embed_kernel(residue_index.astype(jnp.int32),
                         asym_id.astype(jnp.int32),
                         residue_embed, chain_embed)


# speedup vs baseline: 1.0678x; 1.0678x over previous
import functools

import jax
import jax.numpy as jnp
from jax import lax
from jax.experimental import pallas as pl
from jax.experimental.pallas import tpu as pltpu
from jax.experimental.pallas import tpu_sc as plsc

HIDDEN_HALF = 1024
MAX_RES = 4096
MAX_CHAINS = 64
N_TOKENS = 16384

_info = plsc.get_sparse_core_info()
NC = _info.num_cores
NS = _info.num_subcores
L = _info.num_lanes
NW = NC * NS
B_PER_T = N_TOKENS // NW
CHUNK = 16
N_CHUNKS = B_PER_T // CHUNK
NBUF = 4

_mesh = plsc.VectorSubcoreMesh(core_axis_name="c", subcore_axis_name="s")


@functools.partial(
    pl.kernel,
    mesh=_mesh,
    out_type=jax.ShapeDtypeStruct((N_TOKENS, 2 * HIDDEN_HALF), jnp.float32),
    scratch_types=[
        pltpu.VMEM((B_PER_T,), jnp.int32),
        pltpu.VMEM((B_PER_T,), jnp.int32),
        pltpu.VMEM((CHUNK, HIDDEN_HALF), jnp.float32),
        pltpu.VMEM((CHUNK, HIDDEN_HALF), jnp.float32),
        pltpu.VMEM((CHUNK, HIDDEN_HALF), jnp.float32),
        pltpu.VMEM((CHUNK, HIDDEN_HALF), jnp.float32),
        pltpu.VMEM_SHARED((MAX_CHAINS, HIDDEN_HALF), jnp.float32),
        pltpu.SemaphoreType.DMA,
        pltpu.SemaphoreType.DMA,
        pltpu.SemaphoreType.DMA,
    ],
)
def _embed_kernel(res_idx_hbm, chain_idx_hbm, res_tab_hbm, chain_tab_hbm,
                  out_hbm, ridx_v, cidx_v, buf0, buf1, buf2, buf3, chain_sp,
                  gsem, wsem, csem):
    core = lax.axis_index("c")
    sub = lax.axis_index("s")
    wid = sub * NC + core
    base = wid * B_PER_T
    bufs = (buf0, buf1, buf2, buf3)

    @pl.when(sub == 0)
    def _():
        pltpu.sync_copy(chain_tab_hbm, chain_sp)

    pltpu.sync_copy(res_idx_hbm.at[pl.ds(base, B_PER_T)], ridx_v)
    pltpu.sync_copy(chain_idx_hbm.at[pl.ds(base, B_PER_T)], cidx_v)

    def _clip(i, carry):
        sl = pl.ds(i * L, L)
        ridx_v[sl] = jnp.clip(ridx_v[sl] - 1, 0, MAX_RES - 1)
        cidx_v[sl] = jnp.clip(cidx_v[sl], 0, MAX_CHAINS - 1)
        return carry

    lax.fori_loop(0, B_PER_T // L, _clip, 0)
    plsc.subcore_barrier()

    def res_gather(c, buf):
        isl = pl.ds(c * CHUNK, CHUNK)
        return pltpu.async_copy(res_tab_hbm.at[ridx_v.at[isl]], buf, gsem)

    g = [None] * NBUF
    w = [None] * NBUF
    ch = []
    for p in range(min(3, N_CHUNKS)):
        g[p] = res_gather(p, bufs[p])
    for c in range(N_CHUNKS):
        for grp in range(c * CHUNK // L, (c + 1) * CHUNK // L):
            rows = cidx_v[pl.ds(grp * L, L)]
            for lane in range(L):
                t = grp * L + lane
                ch.append(pltpu.async_copy(
                    chain_sp.at[rows[lane]],
                    out_hbm.at[base + t, pl.ds(HIDDEN_HALF, HIDDEN_HALF)],
                    csem))
        cur = c % NBUF
        g[cur].wait()
        if c + 3 < N_CHUNKS:
            nxt = (c + 3) % NBUF
            if w[nxt] is not None:
                w[nxt].wait()
            g[nxt] = res_gather(c + 3, bufs[nxt])
        w[cur] = pltpu.async_copy(
            bufs[cur],
            out_hbm.at[pl.ds(base + c * CHUNK, CHUNK), pl.ds(0, HIDDEN_HALF)],
            wsem)
    for p in range(NBUF):
        if w[p] is not None:
            w[p].wait()
    for h in ch:
        h.wait()


def kernel(residue_index, asym_id, residue_embed, chain_embed):
    return _embed_kernel(residue_index.astype(jnp.int32),
                         asym_id.astype(jnp.int32),
                         residue_embed, chain_embed)


# speedup vs baseline: 1.0914x; 1.0222x over previous
import functools

import jax
import jax.numpy as jnp
from jax import lax
from jax.experimental import pallas as pl
from jax.experimental.pallas import tpu as pltpu
from jax.experimental.pallas import tpu_sc as plsc

HIDDEN_HALF = 1024
MAX_RES = 4096
MAX_CHAINS = 64
N_TOKENS = 16384

_info = plsc.get_sparse_core_info()
NC = _info.num_cores
NS = _info.num_subcores
L = _info.num_lanes
NW = NC * NS
B_PER_T = N_TOKENS // NW
CHUNK = 32
N_CHUNKS = B_PER_T // CHUNK
NBUF = 3

_mesh = plsc.VectorSubcoreMesh(core_axis_name="c", subcore_axis_name="s")


@functools.partial(
    pl.kernel,
    mesh=_mesh,
    out_type=jax.ShapeDtypeStruct((N_TOKENS, 2 * HIDDEN_HALF), jnp.float32),
    scratch_types=[
        pltpu.VMEM((B_PER_T,), jnp.int32),
        pltpu.VMEM((B_PER_T,), jnp.int32),
        pltpu.VMEM((CHUNK, HIDDEN_HALF), jnp.float32),
        pltpu.VMEM((CHUNK, HIDDEN_HALF), jnp.float32),
        pltpu.VMEM((CHUNK, HIDDEN_HALF), jnp.float32),
        pltpu.VMEM_SHARED((MAX_CHAINS, HIDDEN_HALF), jnp.float32),
        pltpu.SemaphoreType.DMA,
        pltpu.SemaphoreType.DMA,
        pltpu.SemaphoreType.DMA,
    ],
)
def _embed_kernel(res_idx_hbm, chain_idx_hbm, res_tab_hbm, chain_tab_hbm,
                  out_hbm, ridx_v, cidx_v, buf0, buf1, buf2, chain_sp,
                  gsem, wsem, csem):
    core = lax.axis_index("c")
    sub = lax.axis_index("s")
    wid = sub * NC + core
    base = wid * B_PER_T
    bufs = (buf0, buf1, buf2)

    @pl.when(sub == 0)
    def _():
        pltpu.sync_copy(chain_tab_hbm, chain_sp)

    pltpu.sync_copy(res_idx_hbm.at[pl.ds(base, B_PER_T)], ridx_v)
    pltpu.sync_copy(chain_idx_hbm.at[pl.ds(base, B_PER_T)], cidx_v)

    def _clip(i, carry):
        sl = pl.ds(i * L, L)
        ridx_v[sl] = jnp.clip(ridx_v[sl] - 1, 0, MAX_RES - 1)
        cidx_v[sl] = jnp.clip(cidx_v[sl], 0, MAX_CHAINS - 1)
        return carry

    lax.fori_loop(0, B_PER_T // L, _clip, 0)
    plsc.subcore_barrier()

    def res_gather(c, buf):
        isl = pl.ds(c * CHUNK, CHUNK)
        return pltpu.async_copy(res_tab_hbm.at[ridx_v.at[isl]], buf, gsem)

    g = [None] * NBUF
    w = [None] * NBUF
    ch = []
    for p in range(min(2, N_CHUNKS)):
        g[p] = res_gather(p, bufs[p])
    for c in range(N_CHUNKS):
        cur = c % NBUF
        g[cur].wait()
        if c + 2 < N_CHUNKS:
            nxt = (c + 2) % NBUF
            if w[nxt] is not None:
                w[nxt].wait()
            g[nxt] = res_gather(c + 2, bufs[nxt])
        w[cur] = pltpu.async_copy(
            bufs[cur],
            out_hbm.at[pl.ds(base + c * CHUNK, CHUNK), pl.ds(0, HIDDEN_HALF)],
            wsem)
        for grp in range(c * CHUNK // L, (c + 1) * CHUNK // L):
            rows = cidx_v[pl.ds(grp * L, L)]
            for lane in range(L):
                t = grp * L + lane
                ch.append(pltpu.async_copy(
                    chain_sp.at[rows[lane]],
                    out_hbm.at[base + t, pl.ds(HIDDEN_HALF, HIDDEN_HALF)],
                    csem))
    for p in range(NBUF):
        if w[p] is not None:
            w[p].wait()
    for h in ch:
        h.wait()


def kernel(residue_index, asym_id, residue_embed, chain_embed):
    return _embed_kernel(residue_index.astype(jnp.int32),
                         asym_id.astype(jnp.int32),
                         residue_embed, chain_embed)
